# baseline (speedup 1.0000x reference)
import functools

import jax
import jax.numpy as jnp
from jax import lax
from jax.experimental import pallas as pl
from jax.experimental.pallas import tpu as pltpu
from jax.experimental.pallas import tpu_sc as plsc

D_MODEL = 64
BATCH = 4096
N_MOTIFS = 200
PAIRS = BATCH * N_MOTIFS // 2
NUM_WORKERS = 32
IDX_MINOR = 128
GATHERS_PER_GROUP = 4
GROUP = GATHERS_PER_GROUP * IDX_MINOR
GROUPS_PER_W = PAIRS // NUM_WORKERS // GROUP


def _sc_lookup(idx2d, pair_table):
    mesh = plsc.VectorSubcoreMesh(core_axis_name="c", subcore_axis_name="s")

    @functools.partial(
        pl.kernel,
        mesh=mesh,
        out_type=jax.ShapeDtypeStruct((PAIRS // IDX_MINOR, IDX_MINOR, 2 * D_MODEL),
                                      jnp.float32),
        scratch_types=[
            pltpu.VMEM((GATHERS_PER_GROUP, IDX_MINOR), jnp.int32),
            pltpu.VMEM((GATHERS_PER_GROUP, IDX_MINOR, 2 * D_MODEL), jnp.float32),
            pltpu.SemaphoreType.DMA,
        ],
    )
    def k(idx_hbm, table_hbm, out_hbm, idx_v, rows_v, sem):
        wid = lax.axis_index("s") * 2 + lax.axis_index("c")
        base = wid * GROUPS_PER_W * GATHERS_PER_GROUP

        def body(g, carry):
            off = base + g * GATHERS_PER_GROUP
            pltpu.sync_copy(idx_hbm.at[pl.ds(off, GATHERS_PER_GROUP)], idx_v)
            handles = []
            for j in range(GATHERS_PER_GROUP):
                handles.append(
                    pltpu.async_copy(table_hbm.at[idx_v.at[j]], rows_v.at[j], sem))
            for h in handles:
                h.wait()
            pltpu.sync_copy(rows_v, out_hbm.at[pl.ds(off, GATHERS_PER_GROUP)])
            return carry

        lax.fori_loop(0, GROUPS_PER_W, body, 0)

    return k(idx2d, pair_table)


def kernel(strands, strand_embed):
    s = strands.reshape(PAIRS, 2).astype(jnp.int32)
    pidx = (s[:, 0] * 2 + s[:, 1]).reshape(PAIRS // IDX_MINOR, IDX_MINOR)
    hi = jnp.repeat(strand_embed, 2, axis=0)
    lo = jnp.tile(strand_embed, (2, 1))
    pair_table = jnp.concatenate([hi, lo], axis=1)
    out = _sc_lookup(pidx, pair_table)
    return out.reshape(BATCH, N_MOTIFS, D_MODEL)


# speedup vs baseline: 4.3569x; 4.3569x over previous
import functools

import jax
import jax.numpy as jnp
from jax import lax
from jax.experimental import pallas as pl
from jax.experimental.pallas import tpu as pltpu
from jax.experimental.pallas import tpu_sc as plsc

D_MODEL = 64
BATCH = 4096
N_MOTIFS = 200
PAIRS = BATCH * N_MOTIFS // 2
NUM_WORKERS = 32
IDX_MINOR = 128
GATHERS_PER_GROUP = 4
GROUP = GATHERS_PER_GROUP * IDX_MINOR
GROUPS_PER_W = PAIRS // NUM_WORKERS // GROUP


def _sc_lookup(idx2d, pair_table):
    mesh = plsc.VectorSubcoreMesh(core_axis_name="c", subcore_axis_name="s")

    @functools.partial(
        pl.kernel,
        mesh=mesh,
        out_type=jax.ShapeDtypeStruct((PAIRS // IDX_MINOR, IDX_MINOR, 2 * D_MODEL),
                                      jnp.float32),
        scratch_types=[
            pltpu.VMEM((GATHERS_PER_GROUP, IDX_MINOR), jnp.int32),
            pltpu.VMEM((GATHERS_PER_GROUP, IDX_MINOR, 2 * D_MODEL), jnp.float32),
            pltpu.SemaphoreType.DMA,
        ],
    )
    def k(idx_hbm, table_hbm, out_hbm, idx_v, rows_v, sem):
        wid = lax.axis_index("s") * 2 + lax.axis_index("c")
        base = wid * GROUPS_PER_W * GATHERS_PER_GROUP

        def body(g, carry):
            off = base + g * GATHERS_PER_GROUP
            pltpu.sync_copy(idx_hbm.at[pl.ds(off, GATHERS_PER_GROUP)], idx_v)
            handles = []
            for j in range(GATHERS_PER_GROUP):
                handles.append(
                    pltpu.async_copy(table_hbm.at[idx_v.at[j]], rows_v.at[j], sem))
            for h in handles:
                h.wait()
            pltpu.sync_copy(rows_v, out_hbm.at[pl.ds(off, GATHERS_PER_GROUP)])
            return carry

        lax.fori_loop(0, GROUPS_PER_W, body, 0)

    return k(idx2d, pair_table)


REPLICAS = 64


def kernel(strands, strand_embed):
    s = strands.reshape(PAIRS, 2).astype(jnp.int32)
    pidx = s[:, 0] * 2 + s[:, 1]
    pidx = pidx + 4 * (jnp.arange(PAIRS, dtype=jnp.int32) % REPLICAS)
    pidx = pidx.reshape(PAIRS // IDX_MINOR, IDX_MINOR)
    hi = jnp.repeat(strand_embed, 2, axis=0)
    lo = jnp.tile(strand_embed, (2, 1))
    pair_table = jnp.concatenate([hi, lo], axis=1)
    pair_table = jnp.tile(pair_table, (REPLICAS, 1))
    out = _sc_lookup(pidx, pair_table)
    return out.reshape(BATCH, N_MOTIFS, D_MODEL)


# speedup vs baseline: 5.5848x; 1.2818x over previous
import functools

import jax
import jax.numpy as jnp
from jax import lax
from jax.experimental import pallas as pl
from jax.experimental.pallas import tpu as pltpu
from jax.experimental.pallas import tpu_sc as plsc

D_MODEL = 64
BATCH = 4096
N_MOTIFS = 200
PACK = 4
ROW_W = PACK * D_MODEL
QUADS = BATCH * N_MOTIFS // PACK
NUM_WORKERS = 32
ROWS_PER_GATHER = 128
GPW = QUADS // NUM_WORKERS // ROWS_PER_GATHER
REPLICAS = 64


def _sc_lookup(idx2d, qtable):
    mesh = plsc.VectorSubcoreMesh(core_axis_name="c", subcore_axis_name="s")

    @functools.partial(
        pl.kernel,
        mesh=mesh,
        out_type=jax.ShapeDtypeStruct(
            (QUADS // ROWS_PER_GATHER, ROWS_PER_GATHER, ROW_W), jnp.float32),
        scratch_types=[
            pltpu.VMEM((2, ROWS_PER_GATHER), jnp.int32),
            pltpu.VMEM((2, ROWS_PER_GATHER, ROW_W), jnp.float32),
            pltpu.SemaphoreType.DMA,
            pltpu.SemaphoreType.DMA,
        ],
    )
    def k(idx_hbm, table_hbm, out_hbm, idx_v, rows_v, sem_g, sem_w):
        wid = lax.axis_index("s") * 2 + lax.axis_index("c")
        base = wid * GPW

        def gather(i, b):
            pltpu.async_copy(table_hbm.at[idx_v.at[b]], rows_v.at[b], sem_g)

        def gather_wait(b):
            pltpu.make_async_copy(table_hbm.at[idx_v.at[b]], rows_v.at[b],
                                  sem_g).wait()

        def writeout(i, b):
            pltpu.async_copy(rows_v.at[b], out_hbm.at[base + i], sem_w)

        def writeout_wait(i, b):
            pltpu.make_async_copy(rows_v.at[b], out_hbm.at[base + i],
                                  sem_w).wait()

        pltpu.sync_copy(idx_hbm.at[base], idx_v.at[0])
        gather(0, 0)

        def body(t, carry):
            for b in (0, 1):
                i = 2 * t + b
                nb = 1 - b

                def refill():
                    writeout_wait(i - 1, nb)
                    pltpu.sync_copy(idx_hbm.at[base + i + 1], idx_v.at[nb])
                    gather(i + 1, nb)

                if b == 0:
                    pl.when(t > 0)(refill)
                    pl.when(t == 0)(lambda: (
                        pltpu.sync_copy(idx_hbm.at[base + 1], idx_v.at[1]),
                        gather(1, 1), None)[-1])
                else:
                    pl.when(t < GPW // 2 - 1)(refill)

                gather_wait(b)
                writeout(i, b)
            return carry

        lax.fori_loop(0, GPW // 2, body, 0)

        writeout_wait(GPW - 2, 0)
        writeout_wait(GPW - 1, 1)

    return k(idx2d, qtable)


def kernel(strands, strand_embed):
    s = strands.reshape(QUADS, PACK).astype(jnp.int32)
    q = s[:, 0] * 8 + s[:, 1] * 4 + s[:, 2] * 2 + s[:, 3]
    q = q + 16 * (jnp.arange(QUADS, dtype=jnp.int32) % REPLICAS)
    q = q.reshape(QUADS // ROWS_PER_GATHER, ROWS_PER_GATHER)
    p = jnp.arange(16)
    qtable = jnp.concatenate([strand_embed[(p >> 3) & 1],
                              strand_embed[(p >> 2) & 1],
                              strand_embed[(p >> 1) & 1],
                              strand_embed[p & 1]], axis=1)
    qtable = jnp.tile(qtable, (REPLICAS, 1))
    out = _sc_lookup(q, qtable)
    return out.reshape(BATCH, N_MOTIFS, D_MODEL)
